# baseline (speedup 1.0000x reference)
import jax
import jax.numpy as jnp
from jax import lax
from jax.experimental import pallas as pl
from jax.experimental.pallas import tpu as pltpu
from jax.experimental.pallas import tpu_sc as plsc

N = 16777216
NW = 32
PW = N // NW
CHUNK = 32768
NCHUNK = PW // CHUNK
UNROLL = 8
NBUF = 3


def _sc_body(b_hbm, x_hbm, o_hbm, bv, xb0, xb1, xb2,
             si0, si1, si2, so0, so1, so2):
    wid = lax.axis_index("s") * 2 + lax.axis_index("c")
    base = wid * PW

    pltpu.sync_copy(b_hbm, bv)
    bvec = bv[...]
    b0 = bvec[0]
    b1 = bvec[1]
    b2 = bvec[2]

    xbufs = (xb0, xb1, xb2)
    isems = (si0, si1, si2)
    osems = (so0, so1, so2)

    in_cp = [None, None, None]
    out_cp = [None, None, None]

    for k in range(2):
        in_cp[k] = pltpu.make_async_copy(
            x_hbm.at[pl.ds(base + k * CHUNK, CHUNK)], xbufs[k], isems[k])
        in_cp[k].start()

    one = jnp.full((16,), 1, jnp.int32)
    two = jnp.full((16,), 2, jnp.int32)
    three = jnp.full((16,), 3, jnp.int32)
    zero = jnp.zeros((16,), jnp.int32)

    for k in range(NCHUNK):
        b = k % NBUF
        if k + 2 < NCHUNK:
            nb = (k + 2) % NBUF
            in_cp[nb] = pltpu.make_async_copy(
                x_hbm.at[pl.ds(base + (k + 2) * CHUNK, CHUNK)],
                xbufs[nb], isems[nb])
            in_cp[nb].start()
        in_cp[b].wait()
        if k >= NBUF:
            out_cp[b].wait()

        xb = xbufs[b]


        out_cp[b] = pltpu.make_async_copy(
            xb, o_hbm.at[pl.ds(base + k * CHUNK, CHUNK)], osems[b])
        out_cp[b].start()

    out_cp[(NCHUNK - 3) % NBUF].wait()
    out_cp[(NCHUNK - 2) % NBUF].wait()
    out_cp[(NCHUNK - 1) % NBUF].wait()


def kernel(values, boundaries):
    bpad = jnp.pad(boundaries, (0, 13))
    run = pl.kernel(
        _sc_body,
        out_type=jax.ShapeDtypeStruct((N,), jnp.float32),
        mesh=plsc.VectorSubcoreMesh(
            core_axis_name="c", subcore_axis_name="s",
            num_cores=2, num_subcores=16),
        scratch_types=[
            pltpu.VMEM((16,), jnp.float32),
            pltpu.VMEM((CHUNK,), jnp.float32),
            pltpu.VMEM((CHUNK,), jnp.float32),
            pltpu.VMEM((CHUNK,), jnp.float32),
            pltpu.SemaphoreType.DMA,
            pltpu.SemaphoreType.DMA,
            pltpu.SemaphoreType.DMA,
            pltpu.SemaphoreType.DMA,
            pltpu.SemaphoreType.DMA,
            pltpu.SemaphoreType.DMA,
        ],
    )
    import jax.numpy as _jnp
    return jax.lax.bitcast_convert_type(run(bpad, values), _jnp.int32)


# speedup vs baseline: 1.2981x; 1.2981x over previous
import jax
import jax.numpy as jnp
from jax import lax
from jax.experimental import pallas as pl
from jax.experimental.pallas import tpu as pltpu
from jax.experimental.pallas import tpu_sc as plsc

N = 16777216
NW = 32
PW = N // NW
CHUNK = 16384
NCHUNK = PW // CHUNK
UNROLL = 16
NBUF = 3


def _sc_body(b_hbm, x_hbm, o_hbm, bv, xb0, xb1, xb2, ob0, ob1, ob2,
             si0, si1, si2, so0, so1, so2):
    wid = lax.axis_index("s") * 2 + lax.axis_index("c")
    base = wid * PW

    pltpu.sync_copy(b_hbm, bv)
    bvec = bv[...]
    b0 = bvec[0]
    b1 = bvec[1]
    b2 = bvec[2]

    xbufs = (xb0, xb1, xb2)
    obufs = (ob0, ob1, ob2)
    isems = (si0, si1, si2)
    osems = (so0, so1, so2)

    in_cp = [None, None, None]
    out_cp = [None, None, None]

    for k in range(2):
        in_cp[k] = pltpu.make_async_copy(
            x_hbm.at[pl.ds(base + k * CHUNK, CHUNK)], xbufs[k], isems[k])
        in_cp[k].start()

    one = jnp.full((16,), 1, jnp.int32)
    two = jnp.full((16,), 2, jnp.int32)
    three = jnp.full((16,), 3, jnp.int32)
    zero = jnp.zeros((16,), jnp.int32)

    for k in range(NCHUNK):
        b = k % NBUF
        if k + 2 < NCHUNK:
            nb = (k + 2) % NBUF
            in_cp[nb] = pltpu.make_async_copy(
                x_hbm.at[pl.ds(base + (k + 2) * CHUNK, CHUNK)],
                xbufs[nb], isems[nb])
            in_cp[nb].start()
        in_cp[b].wait()
        if k >= NBUF:
            out_cp[b].wait()

        xb = xbufs[b]
        ob = obufs[b]

        @plsc.parallel_loop(0, CHUNK, step=16, unroll=UNROLL)
        def inner(i, xb=xb, ob=ob):
            x = xb[pl.ds(i, 16)]
            hi = jnp.where(x >= b2, three, two)
            lo = jnp.where(x >= b0, one, zero)
            ob[pl.ds(i, 16)] = jnp.where(x >= b1, hi, lo)

        out_cp[b] = pltpu.make_async_copy(
            ob, o_hbm.at[pl.ds(base + k * CHUNK, CHUNK)], osems[b])
        out_cp[b].start()

    out_cp[(NCHUNK - 3) % NBUF].wait()
    out_cp[(NCHUNK - 2) % NBUF].wait()
    out_cp[(NCHUNK - 1) % NBUF].wait()


def kernel(values, boundaries):
    bpad = jnp.pad(boundaries, (0, 13))
    run = pl.kernel(
        _sc_body,
        out_type=jax.ShapeDtypeStruct((N,), jnp.int32),
        mesh=plsc.VectorSubcoreMesh(
            core_axis_name="c", subcore_axis_name="s",
            num_cores=2, num_subcores=16),
        scratch_types=[
            pltpu.VMEM((16,), jnp.float32),
            pltpu.VMEM((CHUNK,), jnp.float32),
            pltpu.VMEM((CHUNK,), jnp.float32),
            pltpu.VMEM((CHUNK,), jnp.float32),
            pltpu.VMEM((CHUNK,), jnp.int32),
            pltpu.VMEM((CHUNK,), jnp.int32),
            pltpu.VMEM((CHUNK,), jnp.int32),
            pltpu.SemaphoreType.DMA,
            pltpu.SemaphoreType.DMA,
            pltpu.SemaphoreType.DMA,
            pltpu.SemaphoreType.DMA,
            pltpu.SemaphoreType.DMA,
            pltpu.SemaphoreType.DMA,
        ],
    )
    return run(bpad, values)


# speedup vs baseline: 1.5033x; 1.1581x over previous
import jax
import jax.numpy as jnp
from jax import lax
from jax.experimental import pallas as pl
from jax.experimental.pallas import tpu as pltpu
from jax.experimental.pallas import tpu_sc as plsc

N = 16777216
NW = 32
PW = N // NW
CHUNK = 16384
NCHUNK = PW // CHUNK
OCHUNK = 2 * CHUNK
UNROLL = 8
NBUF = 3


def _sc_body(b_hbm, x_hbm, o_hbm, bv, xb0, xb1, xb2, ob0, ob1,
             si0, si1, si2, so0, so1):
    wid = lax.axis_index("s") * 2 + lax.axis_index("c")
    base = wid * PW

    pltpu.sync_copy(b_hbm, bv)
    bvec = bv[...]
    b0 = bvec[0]
    b1 = bvec[1]
    b2 = bvec[2]

    xbufs = (xb0, xb1, xb2)
    obufs = (ob0, ob1)
    isems = (si0, si1, si2)
    osems = (so0, so1)

    in_cp = [None, None, None]
    out_cp = [None, None]

    for k in range(2):
        in_cp[k] = pltpu.make_async_copy(
            x_hbm.at[pl.ds(base + k * CHUNK, CHUNK)], xbufs[k], isems[k])
        in_cp[k].start()

    one = jnp.full((16,), 1, jnp.int32)
    two = jnp.full((16,), 2, jnp.int32)
    three = jnp.full((16,), 3, jnp.int32)
    zero = jnp.zeros((16,), jnp.int32)

    for k in range(NCHUNK):
        b = k % NBUF
        j = k // 2
        half = k % 2
        if k + 2 < NCHUNK:
            nb = (k + 2) % NBUF
            in_cp[nb] = pltpu.make_async_copy(
                x_hbm.at[pl.ds(base + (k + 2) * CHUNK, CHUNK)],
                xbufs[nb], isems[nb])
            in_cp[nb].start()
        in_cp[b].wait()
        if half == 0 and j >= 2:
            out_cp[j % 2].wait()

        xb = xbufs[b]
        ob = obufs[j % 2]
        off = half * CHUNK

        @plsc.parallel_loop(0, CHUNK, step=16, unroll=UNROLL)
        def inner(i, xb=xb, ob=ob, off=off):
            x = xb[pl.ds(i, 16)]
            hi = jnp.where(x >= b2, three, two)
            lo = jnp.where(x >= b0, one, zero)
            ob[pl.ds(i + off, 16)] = jnp.where(x >= b1, hi, lo)

        if half == 1:
            out_cp[j % 2] = pltpu.make_async_copy(
                ob, o_hbm.at[pl.ds(base + j * OCHUNK, OCHUNK)],
                osems[j % 2])
            out_cp[j % 2].start()

    out_cp[0].wait()
    out_cp[1].wait()


def kernel(values, boundaries):
    bpad = jnp.pad(boundaries, (0, 13))
    run = pl.kernel(
        _sc_body,
        out_type=jax.ShapeDtypeStruct((N,), jnp.int32),
        mesh=plsc.VectorSubcoreMesh(
            core_axis_name="c", subcore_axis_name="s",
            num_cores=2, num_subcores=16),
        scratch_types=[
            pltpu.VMEM((16,), jnp.float32),
            pltpu.VMEM((CHUNK,), jnp.float32),
            pltpu.VMEM((CHUNK,), jnp.float32),
            pltpu.VMEM((CHUNK,), jnp.float32),
            pltpu.VMEM((OCHUNK,), jnp.int32),
            pltpu.VMEM((OCHUNK,), jnp.int32),
            pltpu.SemaphoreType.DMA,
            pltpu.SemaphoreType.DMA,
            pltpu.SemaphoreType.DMA,
            pltpu.SemaphoreType.DMA,
            pltpu.SemaphoreType.DMA,
        ],
    )
    return run(bpad, values)


# speedup vs baseline: 1.5296x; 1.0175x over previous
import jax
import jax.numpy as jnp
from jax import lax
from jax.experimental import pallas as pl
from jax.experimental.pallas import tpu as pltpu
from jax.experimental.pallas import tpu_sc as plsc

N = 16777216
NW = 32
PW = N // NW
CHUNK = 16384
UNROLL = 8
NBUF = 3

_SIZES = [4096, 4096, 8192] + [16384] * 30 + [8192, 4096, 4096]
_OFFS = []
_o = 0
for _s in _SIZES:
    _OFFS.append(_o)
    _o += _s
assert _o == PW
NCH = len(_SIZES)


def _sc_body(b_hbm, x_hbm, o_hbm, bv, xb0, xb1, xb2, ob0, ob1, ob2,
             si0, si1, si2, so0, so1, so2):
    wid = lax.axis_index("s") * 2 + lax.axis_index("c")
    base = wid * PW

    xbufs = (xb0, xb1, xb2)
    obufs = (ob0, ob1, ob2)
    isems = (si0, si1, si2)
    osems = (so0, so1, so2)

    in_cp = [None, None, None]
    out_cp = [None, None, None]

    for k in range(2):
        in_cp[k] = pltpu.make_async_copy(
            x_hbm.at[pl.ds(base + _OFFS[k], _SIZES[k])],
            xbufs[k].at[pl.ds(0, _SIZES[k])], isems[k])
        in_cp[k].start()

    pltpu.sync_copy(b_hbm, bv)
    bvec = bv[...]
    b0 = bvec[0]
    b1 = bvec[1]
    b2 = bvec[2]

    one = jnp.full((16,), 1, jnp.int32)
    two = jnp.full((16,), 2, jnp.int32)
    three = jnp.full((16,), 3, jnp.int32)
    zero = jnp.zeros((16,), jnp.int32)

    for k in range(NCH):
        b = k % NBUF
        if k + 2 < NCH:
            nb = (k + 2) % NBUF
            in_cp[nb] = pltpu.make_async_copy(
                x_hbm.at[pl.ds(base + _OFFS[k + 2], _SIZES[k + 2])],
                xbufs[nb].at[pl.ds(0, _SIZES[k + 2])], isems[nb])
            in_cp[nb].start()
        in_cp[b].wait()
        if k >= NBUF:
            out_cp[b].wait()

        xb = xbufs[b]
        ob = obufs[b]
        sz = _SIZES[k]

        @plsc.parallel_loop(0, sz, step=16, unroll=UNROLL)
        def inner(i, xb=xb, ob=ob):
            x = xb[pl.ds(i, 16)]
            hi = jnp.where(x >= b2, three, two)
            lo = jnp.where(x >= b0, one, zero)
            ob[pl.ds(i, 16)] = jnp.where(x >= b1, hi, lo)

        out_cp[b] = pltpu.make_async_copy(
            ob.at[pl.ds(0, sz)],
            o_hbm.at[pl.ds(base + _OFFS[k], sz)], osems[b])
        out_cp[b].start()

    out_cp[(NCH - 3) % NBUF].wait()
    out_cp[(NCH - 2) % NBUF].wait()
    out_cp[(NCH - 1) % NBUF].wait()


def kernel(values, boundaries):
    bpad = jnp.pad(boundaries, (0, 13))
    run = pl.kernel(
        _sc_body,
        out_type=jax.ShapeDtypeStruct((N,), jnp.int32),
        mesh=plsc.VectorSubcoreMesh(
            core_axis_name="c", subcore_axis_name="s",
            num_cores=2, num_subcores=16),
        scratch_types=[
            pltpu.VMEM((16,), jnp.float32),
            pltpu.VMEM((CHUNK,), jnp.float32),
            pltpu.VMEM((CHUNK,), jnp.float32),
            pltpu.VMEM((CHUNK,), jnp.float32),
            pltpu.VMEM((CHUNK,), jnp.int32),
            pltpu.VMEM((CHUNK,), jnp.int32),
            pltpu.VMEM((CHUNK,), jnp.int32),
            pltpu.SemaphoreType.DMA,
            pltpu.SemaphoreType.DMA,
            pltpu.SemaphoreType.DMA,
            pltpu.SemaphoreType.DMA,
            pltpu.SemaphoreType.DMA,
            pltpu.SemaphoreType.DMA,
        ],
    )
    return run(bpad, values)


# speedup vs baseline: 1.5299x; 1.0002x over previous
import jax
import jax.numpy as jnp
from jax import lax
from jax.experimental import pallas as pl
from jax.experimental.pallas import tpu as pltpu
from jax.experimental.pallas import tpu_sc as plsc

N = 16777216
NW = 32
PW = N // NW
CHUNK = 16384
UNROLL = 8
NBUF = 3

_SIZES = [4096, 4096, 8192] + [16384] * 30 + [8192, 4096, 4096]
_OFFS = []
_o = 0
for _s in _SIZES:
    _OFFS.append(_o)
    _o += _s
assert _o == PW
NCH = len(_SIZES)


def _sc_body(b_hbm, x_hbm, o_hbm, bv, xb0, xb1, xb2, xb3, ob0, ob1, ob2,
             si0, si1, si2, si3, so0, so1, so2):
    wid = lax.axis_index("s") * 2 + lax.axis_index("c")
    base = wid * PW

    xbufs = (xb0, xb1, xb2, xb3)
    obufs = (ob0, ob1, ob2)
    isems = (si0, si1, si2, si3)
    osems = (so0, so1, so2)

    in_cp = [None, None, None, None]
    out_cp = [None, None, None]

    for k in range(3):
        in_cp[k] = pltpu.make_async_copy(
            x_hbm.at[pl.ds(base + _OFFS[k], _SIZES[k])],
            xbufs[k].at[pl.ds(0, _SIZES[k])], isems[k])
        in_cp[k].start()

    pltpu.sync_copy(b_hbm, bv)
    bvec = bv[...]
    b0 = bvec[0]
    b1 = bvec[1]
    b2 = bvec[2]

    one = jnp.full((16,), 1, jnp.int32)
    two = jnp.full((16,), 2, jnp.int32)
    three = jnp.full((16,), 3, jnp.int32)
    zero = jnp.zeros((16,), jnp.int32)

    for k in range(NCH):
        bi = k % 4
        b = k % NBUF
        if k + 3 < NCH:
            nb = (k + 3) % 4
            in_cp[nb] = pltpu.make_async_copy(
                x_hbm.at[pl.ds(base + _OFFS[k + 3], _SIZES[k + 3])],
                xbufs[nb].at[pl.ds(0, _SIZES[k + 3])], isems[nb])
            in_cp[nb].start()
        in_cp[bi].wait()
        if k >= NBUF:
            out_cp[b].wait()

        xb = xbufs[bi]
        ob = obufs[b]
        sz = _SIZES[k]

        @plsc.parallel_loop(0, sz, step=16, unroll=UNROLL)
        def inner(i, xb=xb, ob=ob):
            x = xb[pl.ds(i, 16)]
            hi = jnp.where(x >= b2, three, two)
            lo = jnp.where(x >= b0, one, zero)
            ob[pl.ds(i, 16)] = jnp.where(x >= b1, hi, lo)

        out_cp[b] = pltpu.make_async_copy(
            ob.at[pl.ds(0, sz)],
            o_hbm.at[pl.ds(base + _OFFS[k], sz)], osems[b])
        out_cp[b].start()

    out_cp[(NCH - 3) % NBUF].wait()
    out_cp[(NCH - 2) % NBUF].wait()
    out_cp[(NCH - 1) % NBUF].wait()


def kernel(values, boundaries):
    bpad = jnp.pad(boundaries, (0, 13))
    run = pl.kernel(
        _sc_body,
        out_type=jax.ShapeDtypeStruct((N,), jnp.int32),
        mesh=plsc.VectorSubcoreMesh(
            core_axis_name="c", subcore_axis_name="s",
            num_cores=2, num_subcores=16),
        scratch_types=[
            pltpu.VMEM((16,), jnp.float32),
            pltpu.VMEM((CHUNK,), jnp.float32),
            pltpu.VMEM((CHUNK,), jnp.float32),
            pltpu.VMEM((CHUNK,), jnp.float32),
            pltpu.VMEM((CHUNK,), jnp.float32),
            pltpu.VMEM((CHUNK,), jnp.int32),
            pltpu.VMEM((CHUNK,), jnp.int32),
            pltpu.VMEM((CHUNK,), jnp.int32),
            pltpu.SemaphoreType.DMA,
            pltpu.SemaphoreType.DMA,
            pltpu.SemaphoreType.DMA,
            pltpu.SemaphoreType.DMA,
            pltpu.SemaphoreType.DMA,
            pltpu.SemaphoreType.DMA,
            pltpu.SemaphoreType.DMA,
        ],
    )
    return run(bpad, values)


# speedup vs baseline: 1.5351x; 1.0034x over previous
import jax
import jax.numpy as jnp
from jax import lax
from jax.experimental import pallas as pl
from jax.experimental.pallas import tpu as pltpu
from jax.experimental.pallas import tpu_sc as plsc

N = 16777216
NW = 32
PW = N // NW
CHUNK = 16384
UNROLL = 8
NBUF = 3

_SIZES = [2048, 2048, 4096, 8192] + [16384] * 30 + [8192, 4096, 2048, 2048]
_OFFS = []
_o = 0
for _s in _SIZES:
    _OFFS.append(_o)
    _o += _s
assert _o == PW
NCH = len(_SIZES)


def _sc_body(b_hbm, x_hbm, o_hbm, bv, xb0, xb1, xb2, ob0, ob1, ob2,
             si0, si1, si2, so0, so1, so2):
    wid = lax.axis_index("s") * 2 + lax.axis_index("c")
    base = wid * PW

    xbufs = (xb0, xb1, xb2)
    obufs = (ob0, ob1, ob2)
    isems = (si0, si1, si2)
    osems = (so0, so1, so2)

    in_cp = [None, None, None]
    out_cp = [None, None, None]

    for k in range(2):
        in_cp[k] = pltpu.make_async_copy(
            x_hbm.at[pl.ds(base + _OFFS[k], _SIZES[k])],
            xbufs[k].at[pl.ds(0, _SIZES[k])], isems[k])
        in_cp[k].start()

    pltpu.sync_copy(b_hbm, bv)
    bvec = bv[...]
    b0 = bvec[0]
    b1 = bvec[1]
    b2 = bvec[2]

    one = jnp.full((16,), 1, jnp.int32)
    two = jnp.full((16,), 2, jnp.int32)
    three = jnp.full((16,), 3, jnp.int32)
    zero = jnp.zeros((16,), jnp.int32)

    for k in range(NCH):
        b = k % NBUF
        if k + 2 < NCH:
            nb = (k + 2) % NBUF
            in_cp[nb] = pltpu.make_async_copy(
                x_hbm.at[pl.ds(base + _OFFS[k + 2], _SIZES[k + 2])],
                xbufs[nb].at[pl.ds(0, _SIZES[k + 2])], isems[nb])
            in_cp[nb].start()
        in_cp[b].wait()
        if k >= NBUF:
            out_cp[b].wait()

        xb = xbufs[b]
        ob = obufs[b]
        sz = _SIZES[k]

        @plsc.parallel_loop(0, sz, step=16, unroll=UNROLL)
        def inner(i, xb=xb, ob=ob):
            x = xb[pl.ds(i, 16)]
            hi = jnp.where(x >= b2, three, two)
            lo = jnp.where(x >= b0, one, zero)
            ob[pl.ds(i, 16)] = jnp.where(x >= b1, hi, lo)

        out_cp[b] = pltpu.make_async_copy(
            ob.at[pl.ds(0, sz)],
            o_hbm.at[pl.ds(base + _OFFS[k], sz)], osems[b])
        out_cp[b].start()

    out_cp[(NCH - 3) % NBUF].wait()
    out_cp[(NCH - 2) % NBUF].wait()
    out_cp[(NCH - 1) % NBUF].wait()


def kernel(values, boundaries):
    bpad = jnp.pad(boundaries, (0, 13))
    run = pl.kernel(
        _sc_body,
        out_type=jax.ShapeDtypeStruct((N,), jnp.int32),
        mesh=plsc.VectorSubcoreMesh(
            core_axis_name="c", subcore_axis_name="s",
            num_cores=2, num_subcores=16),
        scratch_types=[
            pltpu.VMEM((16,), jnp.float32),
            pltpu.VMEM((CHUNK,), jnp.float32),
            pltpu.VMEM((CHUNK,), jnp.float32),
            pltpu.VMEM((CHUNK,), jnp.float32),
            pltpu.VMEM((CHUNK,), jnp.int32),
            pltpu.VMEM((CHUNK,), jnp.int32),
            pltpu.VMEM((CHUNK,), jnp.int32),
            pltpu.SemaphoreType.DMA,
            pltpu.SemaphoreType.DMA,
            pltpu.SemaphoreType.DMA,
            pltpu.SemaphoreType.DMA,
            pltpu.SemaphoreType.DMA,
            pltpu.SemaphoreType.DMA,
        ],
    )
    return run(bpad, values)
